# Initial kernel scaffold; baseline (speedup 1.0000x reference)
import jax
import jax.numpy as jnp
from jax.experimental import pallas as pl


def kernel(local_features, edge_index, weight1, weight2, att_weight1, att_weight2):
    raise NotImplementedError("write your pallas kernel here")


# speedup vs baseline: 23.5737x; 23.5737x over previous
import functools

import jax
import jax.numpy as jnp
from jax import lax
from jax.experimental import pallas as pl
from jax.experimental.pallas import tpu as pltpu
from jax.experimental.pallas import tpu_sc as plsc

N = 100000
E = 1600000
IN_DIM = 128
HID = 16
NCLS = 40

NW = 32
NP = 100352
EP = 1605632
EPR = EP // 128
CH = 1024
EW = EP // NW
NCH = EW // CH
BLK = 2048
TRASH = NP - 1

_mesh = plsc.VectorSubcoreMesh(core_axis_name="c", subcore_axis_name="s")
_sc_params = pltpu.CompilerParams(needs_layout_passes=False,
                                  use_tc_tiling_on_sc=False)


def _worker_id():
    return lax.axis_index("s") * 2 + lax.axis_index("c")


def _att_max_body(src_hbm, dst_hbm, s_hbm, d_hbm, att_hbm, mparts_hbm,
                  m_loc, src_v, dst_v, sg_v, dg_v, att_v, sem_s, sem_d):
    wid = _worker_id()

    def init_body(i, c):
        m_loc[pl.ds(i * 16, 16)] = jnp.full((16,), -1e30, jnp.float32)
        return c
    lax.fori_loop(0, NP // 16, init_body, 0)

    def chunk_body(ci, c):
        base = wid * EW + ci * CH
        rb = wid * (EW // 128) + ci * (CH // 128)
        pltpu.sync_copy(src_hbm.at[pl.ds(rb, CH // 128)], src_v)
        pltpu.sync_copy(dst_hbm.at[pl.ds(rb, CH // 128)], dst_v)
        copies = []
        for j in range(CH // 128):
            copies.append(pltpu.async_copy(
                s_hbm.at[src_v.at[j]], sg_v.at[pl.ds(j * 128, 128)], sem_s))
            copies.append(pltpu.async_copy(
                d_hbm.at[dst_v.at[j]], dg_v.at[pl.ds(j * 128, 128)], sem_d))
        for cp in copies:
            cp.wait()

        def vec_body(vi, cc):
            sl = pl.ds(vi * 16, 16)
            a = sg_v[sl] + dg_v[sl]
            a = jnp.maximum(a, a * 0.01)
            att_v[sl] = a
            idx = src_v[vi // 8, pl.ds((vi % 8) * 16, 16)]
            cur = plsc.load_gather(m_loc, (idx,))
            plsc.store_scatter(m_loc, (idx,), jnp.maximum(cur, a))
            chk = plsc.load_gather(m_loc, (idx,))
            bad = a > chk

            def cond(st):
                return st[0]

            def wbody(st):
                _, bad_c, chk_c = st
                plsc.store_scatter(m_loc, (idx,), jnp.maximum(chk_c, a),
                                   mask=bad_c)
                chk2 = plsc.load_gather(m_loc, (idx,))
                bad2 = a > chk2
                return (jnp.any(bad2), bad2, chk2)
            lax.while_loop(cond, wbody, (jnp.any(bad), bad, chk))
            return cc
        lax.fori_loop(0, CH // 16, vec_body, 0)
        pltpu.sync_copy(att_v, att_hbm.at[pl.ds(base, CH)])
        return c
    lax.fori_loop(0, NCH, chunk_body, 0)
    pltpu.sync_copy(m_loc, mparts_hbm.at[pl.ds(wid * NP, NP)])


_att_max = pl.kernel(
    _att_max_body,
    out_type=[jax.ShapeDtypeStruct((EP,), jnp.float32),
              jax.ShapeDtypeStruct((NW * NP,), jnp.float32)],
    mesh=_mesh,
    scratch_types=[pltpu.VMEM((NP,), jnp.float32),
                   pltpu.VMEM((CH // 128, 128), jnp.int32),
                   pltpu.VMEM((CH // 128, 128), jnp.int32),
                   pltpu.VMEM((CH,), jnp.float32),
                   pltpu.VMEM((CH,), jnp.float32),
                   pltpu.VMEM((CH,), jnp.float32),
                   pltpu.SemaphoreType.DMA,
                   pltpu.SemaphoreType.DMA],
    compiler_params=_sc_params,
)


def _combine_body(mparts_hbm, m_hbm, acc_v, tmp_v):
    wid = _worker_id()
    sl_n = NP // NW
    base = wid * sl_n
    pltpu.sync_copy(mparts_hbm.at[pl.ds(base, sl_n)], acc_v)

    def part_body(t, c):
        pltpu.sync_copy(mparts_hbm.at[pl.ds(t * NP + base, sl_n)], tmp_v)

        def vec_body(i, cc):
            sl = pl.ds(i * 16, 16)
            acc_v[sl] = jnp.maximum(acc_v[sl], tmp_v[sl])
            return cc
        lax.fori_loop(0, sl_n // 16, vec_body, 0)
        return c
    lax.fori_loop(1, NW, part_body, 0)
    pltpu.sync_copy(acc_v, m_hbm.at[pl.ds(base, sl_n)])


_combine = pl.kernel(
    _combine_body,
    out_type=jax.ShapeDtypeStruct((NP,), jnp.float32),
    mesh=_mesh,
    scratch_types=[pltpu.VMEM((NP // NW,), jnp.float32),
                   pltpu.VMEM((NP // NW,), jnp.float32)],
    compiler_params=_sc_params,
)


CHB = 512
NCHB = EW // CHB


def _agg_body(src_hbm, dst_hbm, att_hbm, m_hbm, feat_hbm,
              outf_hbm, outs_hbm,
              shared_f, shared_s, src_v, dst_v, att_v, mg_v, e_v,
              feat_v, updf_v, upds_v, zf_v, zs_v, sem_m, sem_f):
    cid = lax.axis_index("c")
    sid = lax.axis_index("s")
    wid = sid * 2 + cid
    rows_per_tile = NP // 16
    lanes = lax.iota(jnp.int32, 16)

    z16 = jnp.zeros((16,), jnp.float32)

    def zf_body(r, c):
        zf_v[r, pl.ds(0, HID)] = z16
        return c
    lax.fori_loop(0, 128, zf_body, 0)

    def zs_body(i, c):
        zs_v[pl.ds(i * 16, 16)] = z16
        return c
    lax.fori_loop(0, 8, zs_body, 0)
    zbase = sid * rows_per_tile

    def zero_body(i, c):
        pltpu.sync_copy(zf_v, shared_f.at[pl.ds(zbase + i * 128, 128)])
        pltpu.sync_copy(zs_v, shared_s.at[pl.ds(zbase + i * 128, 128)])
        return c
    lax.fori_loop(0, rows_per_tile // 128, zero_body, 0)
    plsc.subcore_barrier()

    def chunk_body(ci, c):
        base = wid * EW + ci * CHB
        rb = wid * (EW // 128) + ci * (CHB // 128)
        pltpu.sync_copy(src_hbm.at[pl.ds(rb, CHB // 128)], src_v)
        pltpu.sync_copy(dst_hbm.at[pl.ds(rb, CHB // 128)], dst_v)
        pltpu.sync_copy(att_hbm.at[pl.ds(base, CHB)], att_v)
        copies = []
        for j in range(CHB // 128):
            copies.append(pltpu.async_copy(
                m_hbm.at[src_v.at[j]], mg_v.at[pl.ds(j * 128, 128)], sem_m))
            copies.append(pltpu.async_copy(
                feat_hbm.at[dst_v.at[j]], feat_v.at[pl.ds(j * 128, 128)],
                sem_f))
        for cp in copies:
            cp.wait()

        def exp_body(vi, cc):
            sl = pl.ds(vi * 16, 16)
            e16 = jnp.exp(att_v[sl] - mg_v[sl])
            e_v[sl] = e16
            upds_v[sl] = e16
            return cc
        lax.fori_loop(0, CHB // 16, exp_body, 0)

        def upd_body(vi, cc):
            rows = lanes + vi * 16
            e16 = e_v[pl.ds(vi * 16, 16)]
            for col in range(HID):
                cc16 = jnp.full((16,), col, jnp.int32)
                f = plsc.load_gather(feat_v, (rows, cc16))
                plsc.store_scatter(updf_v, (rows, cc16), f * e16)
            return cc
        lax.fori_loop(0, CHB // 16, upd_body, 0)

        for j in range(CHB // 128):
            pltpu.sync_copy(updf_v.at[pl.ds(j * 128, 128)],
                            shared_f.at[src_v.at[j]], add=True)
            pltpu.sync_copy(upds_v.at[pl.ds(j * 128, 128)],
                            shared_s.at[src_v.at[j]], add=True)
        return c
    lax.fori_loop(0, NCHB, chunk_body, 0)

    plsc.subcore_barrier()
    dbase = sid * rows_per_tile
    pltpu.sync_copy(shared_f.at[pl.ds(dbase, rows_per_tile)],
                    outf_hbm.at[pl.ds(cid * NP + dbase, rows_per_tile)])
    pltpu.sync_copy(shared_s.at[pl.ds(dbase, rows_per_tile)],
                    outs_hbm.at[pl.ds(cid * NP + dbase, rows_per_tile)])


_agg = pl.kernel(
    _agg_body,
    out_type=[jax.ShapeDtypeStruct((2 * NP, HID), jnp.float32),
              jax.ShapeDtypeStruct((2 * NP,), jnp.float32)],
    mesh=_mesh,
    scratch_types=[pltpu.VMEM_SHARED((NP, HID), jnp.float32),
                   pltpu.VMEM_SHARED((NP,), jnp.float32),
                   pltpu.VMEM((CHB // 128, 128), jnp.int32),
                   pltpu.VMEM((CHB // 128, 128), jnp.int32),
                   pltpu.VMEM((CHB,), jnp.float32),
                   pltpu.VMEM((CHB,), jnp.float32),
                   pltpu.VMEM((CHB,), jnp.float32),
                   pltpu.VMEM((CHB, HID), jnp.float32),
                   pltpu.VMEM((CHB, HID), jnp.float32),
                   pltpu.VMEM((CHB,), jnp.float32),
                   pltpu.VMEM((128, HID), jnp.float32),
                   pltpu.VMEM((128,), jnp.float32),
                   pltpu.SemaphoreType.DMA,
                   pltpu.SemaphoreType.DMA],
    compiler_params=_sc_params,
)


def _t1_body(x_ref, w1_ref, aw1_ref, feat_ref, s_ref, d_ref):
    hw = jnp.dot(x_ref[...], w1_ref[...], preferred_element_type=jnp.float32)
    feat_ref[...] = hw
    a = aw1_ref[...]
    s_ref[...] = jnp.dot(hw, a[:HID, :], preferred_element_type=jnp.float32)
    d_ref[...] = jnp.dot(hw, a[HID:, :], preferred_element_type=jnp.float32)


def _t1(xp, w1, aw1):
    return pl.pallas_call(
        _t1_body,
        grid=(NP // BLK,),
        in_specs=[pl.BlockSpec((BLK, IN_DIM), lambda i: (i, 0)),
                  pl.BlockSpec((IN_DIM, HID), lambda i: (0, 0)),
                  pl.BlockSpec((2 * HID, 1), lambda i: (0, 0))],
        out_specs=[pl.BlockSpec((BLK, HID), lambda i: (i, 0)),
                   pl.BlockSpec((BLK, 1), lambda i: (i, 0)),
                   pl.BlockSpec((BLK, 1), lambda i: (i, 0))],
        out_shape=[jax.ShapeDtypeStruct((NP, HID), jnp.float32),
                   jax.ShapeDtypeStruct((NP, 1), jnp.float32),
                   jax.ShapeDtypeStruct((NP, 1), jnp.float32)],
    )(xp, w1, aw1)


def _hidden_from_acc(num, ssum):
    safe = jnp.where(ssum > 0, ssum, 1.0)
    return jnp.where(ssum > 0, num / safe, 0.0)


def _t2_body(pf_ref, ps_ref, w2_ref, aw2_ref, feat_ref, s_ref, d_ref):
    num = pf_ref[0] + pf_ref[1]
    ssum = (ps_ref[0] + ps_ref[1])[:, None]
    h = _hidden_from_acc(num, ssum)
    h = jnp.where(h > 0, h, jnp.exp(jnp.minimum(h, 0.0)) - 1.0)
    feat_ref[...] = h
    w2 = w2_ref[...]
    a = aw2_ref[...]
    wa_top = jnp.dot(w2, a[:NCLS, :], preferred_element_type=jnp.float32)
    wa_bot = jnp.dot(w2, a[NCLS:, :], preferred_element_type=jnp.float32)
    s_ref[...] = jnp.dot(h, wa_top, preferred_element_type=jnp.float32)
    d_ref[...] = jnp.dot(h, wa_bot, preferred_element_type=jnp.float32)


def _t2(accf, accs, w2, aw2):
    return pl.pallas_call(
        _t2_body,
        grid=(NP // BLK,),
        in_specs=[pl.BlockSpec((2, BLK, HID), lambda i: (0, i, 0)),
                  pl.BlockSpec((2, BLK), lambda i: (0, i)),
                  pl.BlockSpec((HID, NCLS), lambda i: (0, 0)),
                  pl.BlockSpec((2 * NCLS, 1), lambda i: (0, 0))],
        out_specs=[pl.BlockSpec((BLK, HID), lambda i: (i, 0)),
                   pl.BlockSpec((BLK, 1), lambda i: (i, 0)),
                   pl.BlockSpec((BLK, 1), lambda i: (i, 0))],
        out_shape=[jax.ShapeDtypeStruct((NP, HID), jnp.float32),
                   jax.ShapeDtypeStruct((NP, 1), jnp.float32),
                   jax.ShapeDtypeStruct((NP, 1), jnp.float32)],
    )(accf, accs, w2, aw2)


def _t3_body(pf_ref, ps_ref, w2_ref, o_ref):
    num = pf_ref[0] + pf_ref[1]
    ssum = (ps_ref[0] + ps_ref[1])[:, None]
    h = _hidden_from_acc(num, ssum)
    o = jnp.dot(h, w2_ref[...], preferred_element_type=jnp.float32)
    m = jnp.max(o, axis=1, keepdims=True)
    lse = jnp.log(jnp.sum(jnp.exp(o - m), axis=1, keepdims=True)) + m
    o_ref[...] = o - lse


def _t3(accf, accs, w2):
    return pl.pallas_call(
        _t3_body,
        grid=(NP // BLK,),
        in_specs=[pl.BlockSpec((2, BLK, HID), lambda i: (0, i, 0)),
                  pl.BlockSpec((2, BLK), lambda i: (0, i)),
                  pl.BlockSpec((HID, NCLS), lambda i: (0, 0))],
        out_specs=pl.BlockSpec((BLK, NCLS), lambda i: (i, 0)),
        out_shape=jax.ShapeDtypeStruct((NP, NCLS), jnp.float32),
    )(accf, accs, w2)


def kernel(local_features, edge_index, weight1, weight2,
           att_weight1, att_weight2):
    src = edge_index[0]
    dst = edge_index[1]
    xp = jnp.pad(local_features, ((0, NP - N), (0, 0)))
    srcp = jnp.pad(src, (0, EP - E), constant_values=TRASH).reshape(EPR, 128)
    dstp = jnp.pad(dst, (0, EP - E), constant_values=TRASH).reshape(EPR, 128)

    feat1, s1, d1 = _t1(xp, weight1, att_weight1)
    att1, mparts1 = _att_max(srcp, dstp, s1.reshape(NP), d1.reshape(NP))
    m1 = _combine(mparts1)
    accf1, accs1 = _agg(srcp, dstp, att1, m1, feat1)

    feat2, s2, d2 = _t2(accf1.reshape(2, NP, HID), accs1.reshape(2, NP),
                        weight2, att_weight2)
    att2, mparts2 = _att_max(srcp, dstp, s2.reshape(NP), d2.reshape(NP))
    m2 = _combine(mparts2)
    accf2, accs2 = _agg(srcp, dstp, att2, m2, feat2)

    out = _t3(accf2.reshape(2, NP, HID), accs2.reshape(2, NP), weight2)
    return out[:N]


# speedup vs baseline: 30.7336x; 1.3037x over previous
import functools

import jax
import jax.numpy as jnp
from jax import lax
from jax.experimental import pallas as pl
from jax.experimental.pallas import tpu as pltpu
from jax.experimental.pallas import tpu_sc as plsc

N = 100000
E = 1600000
IN_DIM = 128
HID = 16
NCLS = 40

NW = 32
NP = 100352
EP = 1605632
EPR = EP // 128
CH = 1024
EW = EP // NW
NCH = EW // CH
BLK = 2048
TRASH = NP - 1

_mesh = plsc.VectorSubcoreMesh(core_axis_name="c", subcore_axis_name="s")
_sc_params = pltpu.CompilerParams(needs_layout_passes=False,
                                  use_tc_tiling_on_sc=False)


def _worker_id():
    return lax.axis_index("s") * 2 + lax.axis_index("c")


CHA = 512
NCHA = EW // CHA
PA = CHA // 128


def _att_max_body(src_hbm, dst_hbm, s_hbm, d_hbm, att_hbm, mparts_hbm,
                  m_loc, src_v, dst_v, sg_v, dg_v, att_v,
                  sem_i0, sem_i1, sem_g0, sem_g1, sem_s0, sem_s1):
    wid = _worker_id()
    sem_i = (sem_i0, sem_i1)
    sem_g = (sem_g0, sem_g1)
    sem_s = (sem_s0, sem_s1)

    def init_body(i, c):
        m_loc[pl.ds(i * 16, 16)] = jnp.full((16,), -1e30, jnp.float32)
        return c
    lax.fori_loop(0, NP // 16, init_body, 0)

    def idx_copies(cj, b):
        rb = wid * (EW // 128) + cj * PA
        return [pltpu.make_async_copy(src_hbm.at[pl.ds(rb, PA)],
                                      src_v.at[b], sem_i[b]),
                pltpu.make_async_copy(dst_hbm.at[pl.ds(rb, PA)],
                                      dst_v.at[b], sem_i[b])]

    def g_copies(b):
        out = []
        for j in range(PA):
            out.append(pltpu.make_async_copy(
                s_hbm.at[src_v.at[b, j]],
                sg_v.at[b, pl.ds(j * 128, 128)], sem_g[b]))
            out.append(pltpu.make_async_copy(
                d_hbm.at[dst_v.at[b, j]],
                dg_v.at[b, pl.ds(j * 128, 128)], sem_g[b]))
        return out

    def s_copies(cj, b):
        base = wid * EW + cj * CHA
        return [pltpu.make_async_copy(att_v.at[b],
                                      att_hbm.at[pl.ds(base, CHA)],
                                      sem_s[b])]

    def fire(copies):
        for cp in copies:
            cp.start()

    def wait(copies):
        for cp in copies:
            cp.wait()

    def compute(b):
        def vec_body(vi, cc):
            sl = pl.ds(vi * 16, 16)
            a = sg_v[b, sl] + dg_v[b, sl]
            a = jnp.maximum(a, a * 0.01)
            att_v[b, sl] = a
            idx = src_v[b, vi // 8, pl.ds((vi % 8) * 16, 16)]
            cur = plsc.load_gather(m_loc, (idx,))
            plsc.store_scatter(m_loc, (idx,), jnp.maximum(cur, a))
            chk = plsc.load_gather(m_loc, (idx,))
            bad = a > chk

            def cond(st):
                return st[0]

            def wbody(st):
                _, bad_c, chk_c = st
                plsc.store_scatter(m_loc, (idx,), jnp.maximum(chk_c, a),
                                   mask=bad_c)
                chk2 = plsc.load_gather(m_loc, (idx,))
                bad2 = a > chk2
                return (jnp.any(bad2), bad2, chk2)
            lax.while_loop(cond, wbody, (jnp.any(bad), bad, chk))
            return cc
        lax.fori_loop(0, CHA // 16, vec_body, 0)

    def sub_iter(k, b, drain_s, fire_next_idx):
        wait(idx_copies(k + 1, 1 - b))
        fire(g_copies(1 - b))
        wait(g_copies(b))
        if drain_s:
            wait(s_copies(k - 2, b))
        compute(b)
        if fire_next_idx:
            fire(idx_copies(k + 2, b))
        fire(s_copies(k, b))

    fire(idx_copies(0, 0))
    fire(idx_copies(1, 1))
    wait(idx_copies(0, 0))
    fire(g_copies(0))
    sub_iter(0, 0, False, True)
    sub_iter(1, 1, False, True)

    def pair_body(g, c):
        k0 = 2 + g * 2
        sub_iter(k0, 0, True, True)
        sub_iter(k0 + 1, 1, True, True)
        return c
    lax.fori_loop(0, (NCHA - 4) // 2, pair_body, 0)

    sub_iter(NCHA - 2, 0, True, False)
    wait(g_copies(1))
    wait(s_copies(NCHA - 3, 1))
    compute(1)
    fire(s_copies(NCHA - 1, 1))
    wait(s_copies(NCHA - 2, 0))
    wait(s_copies(NCHA - 1, 1))

    pltpu.sync_copy(m_loc, mparts_hbm.at[pl.ds(wid * NP, NP)])


_att_max = pl.kernel(
    _att_max_body,
    out_type=[jax.ShapeDtypeStruct((EP,), jnp.float32),
              jax.ShapeDtypeStruct((NW * NP,), jnp.float32)],
    mesh=_mesh,
    scratch_types=[pltpu.VMEM((NP,), jnp.float32),
                   pltpu.VMEM((2, PA, 128), jnp.int32),
                   pltpu.VMEM((2, PA, 128), jnp.int32),
                   pltpu.VMEM((2, CHA), jnp.float32),
                   pltpu.VMEM((2, CHA), jnp.float32),
                   pltpu.VMEM((2, CHA), jnp.float32),
                   pltpu.SemaphoreType.DMA,
                   pltpu.SemaphoreType.DMA,
                   pltpu.SemaphoreType.DMA,
                   pltpu.SemaphoreType.DMA,
                   pltpu.SemaphoreType.DMA,
                   pltpu.SemaphoreType.DMA],
    compiler_params=_sc_params,
)


def _combine_body(mparts_hbm, m_hbm, acc_v, tmp_v):
    wid = _worker_id()
    sl_n = NP // NW
    base = wid * sl_n
    pltpu.sync_copy(mparts_hbm.at[pl.ds(base, sl_n)], acc_v)

    def part_body(t, c):
        pltpu.sync_copy(mparts_hbm.at[pl.ds(t * NP + base, sl_n)], tmp_v)

        def vec_body(i, cc):
            sl = pl.ds(i * 16, 16)
            acc_v[sl] = jnp.maximum(acc_v[sl], tmp_v[sl])
            return cc
        lax.fori_loop(0, sl_n // 16, vec_body, 0)
        return c
    lax.fori_loop(1, NW, part_body, 0)
    pltpu.sync_copy(acc_v, m_hbm.at[pl.ds(base, sl_n)])


_combine = pl.kernel(
    _combine_body,
    out_type=jax.ShapeDtypeStruct((NP,), jnp.float32),
    mesh=_mesh,
    scratch_types=[pltpu.VMEM((NP // NW,), jnp.float32),
                   pltpu.VMEM((NP // NW,), jnp.float32)],
    compiler_params=_sc_params,
)


CHB = 256
NCHB = EW // CHB
PB = CHB // 128


def _agg_body(src_hbm, dst_hbm, att_hbm, m_hbm, feat_hbm,
              outf_hbm, outs_hbm,
              shared_f, shared_s, src_v, dst_v, att_v, mg_v, e_v,
              feat_v, updf_v, sidx_v, zf_v, zs_v,
              sem_i0, sem_i1, sem_g0, sem_g1, sem_s0, sem_s1):
    cid = lax.axis_index("c")
    sid = lax.axis_index("s")
    wid = sid * 2 + cid
    rows_per_tile = NP // 16
    lanes = lax.iota(jnp.int32, 16)
    sem_i = (sem_i0, sem_i1)
    sem_g = (sem_g0, sem_g1)
    sem_s = (sem_s0, sem_s1)

    z16 = jnp.zeros((16,), jnp.float32)

    def zf_body(r, c):
        zf_v[r, pl.ds(0, HID)] = z16
        return c
    lax.fori_loop(0, 128, zf_body, 0)

    def zs_body(i, c):
        zs_v[pl.ds(i * 16, 16)] = z16
        return c
    lax.fori_loop(0, 8, zs_body, 0)
    zbase = sid * rows_per_tile

    def zero_body(i, c):
        pltpu.sync_copy(zf_v, shared_f.at[pl.ds(zbase + i * 128, 128)])
        pltpu.sync_copy(zs_v, shared_s.at[pl.ds(zbase + i * 128, 128)])
        return c
    lax.fori_loop(0, rows_per_tile // 128, zero_body, 0)
    plsc.subcore_barrier()

    def idx_copies(cj, b):
        rb = wid * (EW // 128) + cj * PB
        base = wid * EW + cj * CHB
        return [pltpu.make_async_copy(src_hbm.at[pl.ds(rb, PB)],
                                      src_v.at[b], sem_i[b]),
                pltpu.make_async_copy(dst_hbm.at[pl.ds(rb, PB)],
                                      dst_v.at[b], sem_i[b]),
                pltpu.make_async_copy(att_hbm.at[pl.ds(base, CHB)],
                                      att_v.at[b], sem_i[b])]

    def g_copies(b):
        out = []
        for j in range(PB):
            out.append(pltpu.make_async_copy(
                m_hbm.at[src_v.at[b, j]],
                mg_v.at[b, pl.ds(j * 128, 128)], sem_g[b]))
            out.append(pltpu.make_async_copy(
                feat_hbm.at[dst_v.at[b, j]],
                feat_v.at[b, pl.ds(j * 128, 128)], sem_g[b]))
        return out

    def s_copies(b):
        out = []
        for j in range(PB):
            out.append(pltpu.make_async_copy(
                updf_v.at[b, pl.ds(j * 128, 128)],
                shared_f.at[sidx_v.at[b, j]], sem_s[b]))
            out.append(pltpu.make_async_copy(
                e_v.at[b, pl.ds(j * 128, 128)],
                shared_s.at[sidx_v.at[b, j]], sem_s[b]))
        return out

    def fire(copies, add=False):
        for cp in copies:
            cp.start(add=add)

    def wait(copies):
        for cp in copies:
            cp.wait()

    def compute(b):
        def exp_body(vi, cc):
            sl = pl.ds(vi * 16, 16)
            e_v[b, sl] = jnp.exp(att_v[b, sl] - mg_v[b, sl])
            return cc
        lax.fori_loop(0, CHB // 16, exp_body, 0)

        def cpidx_body(vi, cc):
            sl = pl.ds(vi * 16, 16)
            for j in range(PB):
                sidx_v[b, j, sl] = src_v[b, j, sl]
            return cc
        lax.fori_loop(0, 8, cpidx_body, 0)

        fv = feat_v.at[b]
        uv = updf_v.at[b]

        def upd_body(vi, cc):
            rows = lanes + vi * 16
            e16 = e_v[b, pl.ds(vi * 16, 16)]
            for col in range(HID):
                cc16 = jnp.full((16,), col, jnp.int32)
                f = plsc.load_gather(fv, (rows, cc16))
                plsc.store_scatter(uv, (rows, cc16), f * e16)
            return cc
        lax.fori_loop(0, CHB // 16, upd_body, 0)

    def sub_iter(k, b, drain_s, fire_next_idx):
        wait(idx_copies(k + 1, 1 - b))
        fire(g_copies(1 - b))
        wait(g_copies(b))
        if drain_s:
            wait(s_copies(b))
        compute(b)
        if fire_next_idx:
            fire(idx_copies(k + 2, b))
        fire(s_copies(b), add=True)

    fire(idx_copies(0, 0))
    fire(idx_copies(1, 1))
    wait(idx_copies(0, 0))
    fire(g_copies(0))
    sub_iter(0, 0, False, True)
    sub_iter(1, 1, False, True)

    def pair_body(g, c):
        k0 = 2 + g * 2
        sub_iter(k0, 0, True, True)
        sub_iter(k0 + 1, 1, True, True)
        return c
    lax.fori_loop(0, (NCHB - 4) // 2, pair_body, 0)

    sub_iter(NCHB - 2, 0, True, False)
    wait(g_copies(1))
    wait(s_copies(1))
    compute(1)
    fire(s_copies(1), add=True)
    wait(s_copies(0))
    wait(s_copies(1))

    plsc.subcore_barrier()
    dbase = sid * rows_per_tile
    pltpu.sync_copy(shared_f.at[pl.ds(dbase, rows_per_tile)],
                    outf_hbm.at[pl.ds(cid * NP + dbase, rows_per_tile)])
    pltpu.sync_copy(shared_s.at[pl.ds(dbase, rows_per_tile)],
                    outs_hbm.at[pl.ds(cid * NP + dbase, rows_per_tile)])


_agg = pl.kernel(
    _agg_body,
    out_type=[jax.ShapeDtypeStruct((2 * NP, HID), jnp.float32),
              jax.ShapeDtypeStruct((2 * NP,), jnp.float32)],
    mesh=_mesh,
    scratch_types=[pltpu.VMEM_SHARED((NP, HID), jnp.float32),
                   pltpu.VMEM_SHARED((NP,), jnp.float32),
                   pltpu.VMEM((2, PB, 128), jnp.int32),
                   pltpu.VMEM((2, PB, 128), jnp.int32),
                   pltpu.VMEM((2, CHB), jnp.float32),
                   pltpu.VMEM((2, CHB), jnp.float32),
                   pltpu.VMEM((2, CHB), jnp.float32),
                   pltpu.VMEM((2, CHB, HID), jnp.float32),
                   pltpu.VMEM((2, CHB, HID), jnp.float32),
                   pltpu.VMEM((2, PB, 128), jnp.int32),
                   pltpu.VMEM((128, HID), jnp.float32),
                   pltpu.VMEM((128,), jnp.float32),
                   pltpu.SemaphoreType.DMA,
                   pltpu.SemaphoreType.DMA,
                   pltpu.SemaphoreType.DMA,
                   pltpu.SemaphoreType.DMA,
                   pltpu.SemaphoreType.DMA,
                   pltpu.SemaphoreType.DMA],
    compiler_params=_sc_params,
)


def _t1_body(x_ref, w1_ref, aw1_ref, feat_ref, s_ref, d_ref):
    hw = jnp.dot(x_ref[...], w1_ref[...], preferred_element_type=jnp.float32)
    feat_ref[...] = hw
    a = aw1_ref[...]
    s_ref[...] = jnp.dot(hw, a[:HID, :], preferred_element_type=jnp.float32)
    d_ref[...] = jnp.dot(hw, a[HID:, :], preferred_element_type=jnp.float32)


def _t1(xp, w1, aw1):
    return pl.pallas_call(
        _t1_body,
        grid=(NP // BLK,),
        in_specs=[pl.BlockSpec((BLK, IN_DIM), lambda i: (i, 0)),
                  pl.BlockSpec((IN_DIM, HID), lambda i: (0, 0)),
                  pl.BlockSpec((2 * HID, 1), lambda i: (0, 0))],
        out_specs=[pl.BlockSpec((BLK, HID), lambda i: (i, 0)),
                   pl.BlockSpec((BLK, 1), lambda i: (i, 0)),
                   pl.BlockSpec((BLK, 1), lambda i: (i, 0))],
        out_shape=[jax.ShapeDtypeStruct((NP, HID), jnp.float32),
                   jax.ShapeDtypeStruct((NP, 1), jnp.float32),
                   jax.ShapeDtypeStruct((NP, 1), jnp.float32)],
    )(xp, w1, aw1)


def _hidden_from_acc(num, ssum):
    safe = jnp.where(ssum > 0, ssum, 1.0)
    return jnp.where(ssum > 0, num / safe, 0.0)


def _t2_body(pf_ref, ps_ref, w2_ref, aw2_ref, feat_ref, s_ref, d_ref):
    num = pf_ref[0] + pf_ref[1]
    ssum = (ps_ref[0] + ps_ref[1])[:, None]
    h = _hidden_from_acc(num, ssum)
    h = jnp.where(h > 0, h, jnp.exp(jnp.minimum(h, 0.0)) - 1.0)
    feat_ref[...] = h
    w2 = w2_ref[...]
    a = aw2_ref[...]
    wa_top = jnp.dot(w2, a[:NCLS, :], preferred_element_type=jnp.float32)
    wa_bot = jnp.dot(w2, a[NCLS:, :], preferred_element_type=jnp.float32)
    s_ref[...] = jnp.dot(h, wa_top, preferred_element_type=jnp.float32)
    d_ref[...] = jnp.dot(h, wa_bot, preferred_element_type=jnp.float32)


def _t2(accf, accs, w2, aw2):
    return pl.pallas_call(
        _t2_body,
        grid=(NP // BLK,),
        in_specs=[pl.BlockSpec((2, BLK, HID), lambda i: (0, i, 0)),
                  pl.BlockSpec((2, BLK), lambda i: (0, i)),
                  pl.BlockSpec((HID, NCLS), lambda i: (0, 0)),
                  pl.BlockSpec((2 * NCLS, 1), lambda i: (0, 0))],
        out_specs=[pl.BlockSpec((BLK, HID), lambda i: (i, 0)),
                   pl.BlockSpec((BLK, 1), lambda i: (i, 0)),
                   pl.BlockSpec((BLK, 1), lambda i: (i, 0))],
        out_shape=[jax.ShapeDtypeStruct((NP, HID), jnp.float32),
                   jax.ShapeDtypeStruct((NP, 1), jnp.float32),
                   jax.ShapeDtypeStruct((NP, 1), jnp.float32)],
    )(accf, accs, w2, aw2)


def _t3_body(pf_ref, ps_ref, w2_ref, o_ref):
    num = pf_ref[0] + pf_ref[1]
    ssum = (ps_ref[0] + ps_ref[1])[:, None]
    h = _hidden_from_acc(num, ssum)
    o = jnp.dot(h, w2_ref[...], preferred_element_type=jnp.float32)
    m = jnp.max(o, axis=1, keepdims=True)
    lse = jnp.log(jnp.sum(jnp.exp(o - m), axis=1, keepdims=True)) + m
    o_ref[...] = o - lse


def _t3(accf, accs, w2):
    return pl.pallas_call(
        _t3_body,
        grid=(NP // BLK,),
        in_specs=[pl.BlockSpec((2, BLK, HID), lambda i: (0, i, 0)),
                  pl.BlockSpec((2, BLK), lambda i: (0, i)),
                  pl.BlockSpec((HID, NCLS), lambda i: (0, 0))],
        out_specs=pl.BlockSpec((BLK, NCLS), lambda i: (i, 0)),
        out_shape=jax.ShapeDtypeStruct((NP, NCLS), jnp.float32),
    )(accf, accs, w2)


def kernel(local_features, edge_index, weight1, weight2,
           att_weight1, att_weight2):
    src = edge_index[0]
    dst = edge_index[1]
    xp = jnp.pad(local_features, ((0, NP - N), (0, 0)))
    srcp = jnp.pad(src, (0, EP - E), constant_values=TRASH).reshape(EPR, 128)
    dstp = jnp.pad(dst, (0, EP - E), constant_values=TRASH).reshape(EPR, 128)

    feat1, s1, d1 = _t1(xp, weight1, att_weight1)
    att1, mparts1 = _att_max(srcp, dstp, s1.reshape(NP), d1.reshape(NP))
    m1 = _combine(mparts1)
    accf1, accs1 = _agg(srcp, dstp, att1, m1, feat1)

    feat2, s2, d2 = _t2(accf1.reshape(2, NP, HID), accs1.reshape(2, NP),
                        weight2, att_weight2)
    att2, mparts2 = _att_max(srcp, dstp, s2.reshape(NP), d2.reshape(NP))
    m2 = _combine(mparts2)
    accf2, accs2 = _agg(srcp, dstp, att2, m2, feat2)

    out = _t3(accf2.reshape(2, NP, HID), accs2.reshape(2, NP), weight2)
    return out[:N]


# speedup vs baseline: 34.7442x; 1.1305x over previous
import functools

import jax
import jax.numpy as jnp
from jax import lax
from jax.experimental import pallas as pl
from jax.experimental.pallas import tpu as pltpu
from jax.experimental.pallas import tpu_sc as plsc

N = 100000
E = 1600000
IN_DIM = 128
HID = 16
NCLS = 40

NW = 32
NP = 100352
EP = 1605632
EPR = EP // 128
CH = 1024
EW = EP // NW
NCH = EW // CH
BLK = 2048
TRASH = NP - 1

_mesh = plsc.VectorSubcoreMesh(core_axis_name="c", subcore_axis_name="s")
_sc_params = pltpu.CompilerParams(needs_layout_passes=False,
                                  use_tc_tiling_on_sc=False)


def _worker_id():
    return lax.axis_index("s") * 2 + lax.axis_index("c")


CHA = 512
NCHA = EW // CHA
PA = CHA // 128


def _att_max_body(src_hbm, dst_hbm, s_hbm, d_hbm, att_hbm, mparts_hbm,
                  m_loc, src_v, dst_v, sg_v, dg_v, att_v, sidx_v,
                  sem_i0, sem_i1, sem_g0, sem_g1, sem_s0, sem_s1):
    wid = _worker_id()
    sem_i = (sem_i0, sem_i1)
    sem_g = (sem_g0, sem_g1)
    sem_s = (sem_s0, sem_s1)

    def init_body(i, c):
        m_loc[pl.ds(i * 16, 16)] = jnp.full((16,), -1e30, jnp.float32)
        return c
    lax.fori_loop(0, NP // 16, init_body, 0)

    def idx_copies(cj, b):
        rb = wid * (EW // 128) + cj * PA
        return [pltpu.make_async_copy(src_hbm.at[pl.ds(rb, PA)],
                                      src_v.at[b], sem_i[b]),
                pltpu.make_async_copy(dst_hbm.at[pl.ds(rb, PA)],
                                      dst_v.at[b], sem_i[b])]

    def g_copies(b):
        out = []
        for j in range(PA):
            out.append(pltpu.make_async_copy(
                s_hbm.at[src_v.at[b, j]],
                sg_v.at[b, pl.ds(j * 128, 128)], sem_g[b]))
            out.append(pltpu.make_async_copy(
                d_hbm.at[dst_v.at[b, j]],
                dg_v.at[b, pl.ds(j * 128, 128)], sem_g[b]))
        return out

    def s_copies(cj, b):
        base = wid * EW + cj * CHA
        return [pltpu.make_async_copy(att_v.at[b],
                                      att_hbm.at[pl.ds(base, CHA)],
                                      sem_s[b])]

    def fire(copies):
        for cp in copies:
            cp.start()

    def wait(copies):
        for cp in copies:
            cp.wait()

    def compute_pre(b):
        def att_body(vi, cc):
            sl = pl.ds(vi * 16, 16)
            a = sg_v[b, sl] + dg_v[b, sl]
            a = jnp.maximum(a, a * 0.01)
            att_v[b, sl] = a
            sidx_v[b, sl] = src_v[b, vi // 8, pl.ds((vi % 8) * 16, 16)]
            return cc
        lax.fori_loop(0, CHA // 16, att_body, 0)

    def compute_rmw(b):
        def vec_body(vi, cc):
            sl = pl.ds(vi * 16, 16)
            a = att_v[b, sl]
            idx = sidx_v[b, sl]
            cur = plsc.load_gather(m_loc, (idx,))
            plsc.store_scatter(m_loc, (idx,), jnp.maximum(cur, a))
            chk = plsc.load_gather(m_loc, (idx,))
            bad = a > chk

            def cond(st):
                return st[0]

            def wbody(st):
                _, bad_c, chk_c = st
                plsc.store_scatter(m_loc, (idx,), jnp.maximum(chk_c, a),
                                   mask=bad_c)
                chk2 = plsc.load_gather(m_loc, (idx,))
                bad2 = a > chk2
                return (jnp.any(bad2), bad2, chk2)
            lax.while_loop(cond, wbody, (jnp.any(bad), bad, chk))
            return cc
        lax.fori_loop(0, CHA // 16, vec_body, 0)

    def sub_iter(k, b, drain_s, fire_next_idx):
        wait(idx_copies(k + 1, 1 - b))
        fire(g_copies(1 - b))
        wait(g_copies(b))
        if drain_s:
            wait(s_copies(k - 2, b))
        compute_pre(b)
        if fire_next_idx:
            fire(idx_copies(k + 2, b))
        compute_rmw(b)
        fire(s_copies(k, b))

    fire(idx_copies(0, 0))
    fire(idx_copies(1, 1))
    wait(idx_copies(0, 0))
    fire(g_copies(0))
    sub_iter(0, 0, False, True)
    sub_iter(1, 1, False, True)

    def pair_body(g, c):
        k0 = 2 + g * 2
        sub_iter(k0, 0, True, True)
        sub_iter(k0 + 1, 1, True, True)
        return c
    lax.fori_loop(0, (NCHA - 4) // 2, pair_body, 0)

    sub_iter(NCHA - 2, 0, True, False)
    wait(g_copies(1))
    wait(s_copies(NCHA - 3, 1))
    compute_pre(1)
    compute_rmw(1)
    fire(s_copies(NCHA - 1, 1))
    wait(s_copies(NCHA - 2, 0))
    wait(s_copies(NCHA - 1, 1))

    pltpu.sync_copy(m_loc, mparts_hbm.at[pl.ds(wid * NP, NP)])


_att_max = pl.kernel(
    _att_max_body,
    out_type=[jax.ShapeDtypeStruct((EP,), jnp.float32),
              jax.ShapeDtypeStruct((NW * NP,), jnp.float32)],
    mesh=_mesh,
    scratch_types=[pltpu.VMEM((NP,), jnp.float32),
                   pltpu.VMEM((2, PA, 128), jnp.int32),
                   pltpu.VMEM((2, PA, 128), jnp.int32),
                   pltpu.VMEM((2, CHA), jnp.float32),
                   pltpu.VMEM((2, CHA), jnp.float32),
                   pltpu.VMEM((2, CHA), jnp.float32),
                   pltpu.VMEM((2, CHA), jnp.int32),
                   pltpu.SemaphoreType.DMA,
                   pltpu.SemaphoreType.DMA,
                   pltpu.SemaphoreType.DMA,
                   pltpu.SemaphoreType.DMA,
                   pltpu.SemaphoreType.DMA,
                   pltpu.SemaphoreType.DMA],
    compiler_params=_sc_params,
)


def _combine_body(mparts_hbm, m_hbm, acc_v, tmp_v):
    wid = _worker_id()
    sl_n = NP // NW
    base = wid * sl_n
    pltpu.sync_copy(mparts_hbm.at[pl.ds(base, sl_n)], acc_v)

    def part_body(t, c):
        pltpu.sync_copy(mparts_hbm.at[pl.ds(t * NP + base, sl_n)], tmp_v)

        def vec_body(i, cc):
            sl = pl.ds(i * 16, 16)
            acc_v[sl] = jnp.maximum(acc_v[sl], tmp_v[sl])
            return cc
        lax.fori_loop(0, sl_n // 16, vec_body, 0)
        return c
    lax.fori_loop(1, NW, part_body, 0)
    pltpu.sync_copy(acc_v, m_hbm.at[pl.ds(base, sl_n)])


_combine = pl.kernel(
    _combine_body,
    out_type=jax.ShapeDtypeStruct((NP,), jnp.float32),
    mesh=_mesh,
    scratch_types=[pltpu.VMEM((NP // NW,), jnp.float32),
                   pltpu.VMEM((NP // NW,), jnp.float32)],
    compiler_params=_sc_params,
)


CHB = 256
NCHB = EW // CHB
PB = CHB // 128


def _agg_body(src_hbm, dst_hbm, att_hbm, m_hbm, feat_hbm,
              outf_hbm, outs_hbm,
              shared_f, shared_s, src_v, dst_v, att_v, mg_v, e_v,
              feat_v, updf_v, sidx_v, zf_v, zs_v,
              sem_i0, sem_i1, sem_g0, sem_g1, sem_s0, sem_s1):
    cid = lax.axis_index("c")
    sid = lax.axis_index("s")
    wid = sid * 2 + cid
    rows_per_tile = NP // 16
    lanes = lax.iota(jnp.int32, 16)
    sem_i = (sem_i0, sem_i1)
    sem_g = (sem_g0, sem_g1)
    sem_s = (sem_s0, sem_s1)

    z16 = jnp.zeros((16,), jnp.float32)

    def zf_body(r, c):
        zf_v[r, pl.ds(0, HID)] = z16
        return c
    lax.fori_loop(0, 128, zf_body, 0)

    def zs_body(i, c):
        zs_v[pl.ds(i * 16, 16)] = z16
        return c
    lax.fori_loop(0, 8, zs_body, 0)
    zbase = sid * rows_per_tile

    def zero_body(i, c):
        pltpu.sync_copy(zf_v, shared_f.at[pl.ds(zbase + i * 128, 128)])
        pltpu.sync_copy(zs_v, shared_s.at[pl.ds(zbase + i * 128, 128)])
        return c
    lax.fori_loop(0, rows_per_tile // 128, zero_body, 0)
    plsc.subcore_barrier()

    def idx_copies(cj, b):
        rb = wid * (EW // 128) + cj * PB
        base = wid * EW + cj * CHB
        return [pltpu.make_async_copy(src_hbm.at[pl.ds(rb, PB)],
                                      src_v.at[b], sem_i[b]),
                pltpu.make_async_copy(dst_hbm.at[pl.ds(rb, PB)],
                                      dst_v.at[b], sem_i[b]),
                pltpu.make_async_copy(att_hbm.at[pl.ds(base, CHB)],
                                      att_v.at[b], sem_i[b])]

    def g_copies(b):
        out = []
        for j in range(PB):
            out.append(pltpu.make_async_copy(
                m_hbm.at[src_v.at[b, j]],
                mg_v.at[b, pl.ds(j * 128, 128)], sem_g[b]))
            out.append(pltpu.make_async_copy(
                feat_hbm.at[dst_v.at[b, j]],
                feat_v.at[b, pl.ds(j * 128, 128)], sem_g[b]))
        return out

    def s_copies(b):
        out = []
        for j in range(PB):
            out.append(pltpu.make_async_copy(
                updf_v.at[b, pl.ds(j * 128, 128)],
                shared_f.at[sidx_v.at[b, j]], sem_s[b]))
            out.append(pltpu.make_async_copy(
                e_v.at[b, pl.ds(j * 128, 128)],
                shared_s.at[sidx_v.at[b, j]], sem_s[b]))
        return out

    def fire(copies, add=False):
        for cp in copies:
            cp.start(add=add)

    def wait(copies):
        for cp in copies:
            cp.wait()

    def compute_pre(b):
        def exp_body(vi, cc):
            sl = pl.ds(vi * 16, 16)
            e_v[b, sl] = jnp.exp(att_v[b, sl] - mg_v[b, sl])
            return cc
        lax.fori_loop(0, CHB // 16, exp_body, 0)

        def cpidx_body(vi, cc):
            sl = pl.ds(vi * 16, 16)
            for j in range(PB):
                sidx_v[b, j, sl] = src_v[b, j, sl]
            return cc
        lax.fori_loop(0, 8, cpidx_body, 0)

    def compute_upd(b):
        fv = feat_v.at[b]
        uv = updf_v.at[b]

        def upd_body(vi, cc):
            rows = lanes + vi * 16
            e16 = e_v[b, pl.ds(vi * 16, 16)]
            for col in range(HID):
                cc16 = jnp.full((16,), col, jnp.int32)
                f = plsc.load_gather(fv, (rows, cc16))
                plsc.store_scatter(uv, (rows, cc16), f * e16)
            return cc
        lax.fori_loop(0, CHB // 16, upd_body, 0)

    def sub_iter(k, b, drain_s, fire_next_idx):
        wait(idx_copies(k + 1, 1 - b))
        fire(g_copies(1 - b))
        wait(g_copies(b))
        if drain_s:
            wait(s_copies(b))
        compute_pre(b)
        if fire_next_idx:
            fire(idx_copies(k + 2, b))
        compute_upd(b)
        fire(s_copies(b), add=True)

    fire(idx_copies(0, 0))
    fire(idx_copies(1, 1))
    wait(idx_copies(0, 0))
    fire(g_copies(0))
    sub_iter(0, 0, False, True)
    sub_iter(1, 1, False, True)

    def pair_body(g, c):
        k0 = 2 + g * 2
        sub_iter(k0, 0, True, True)
        sub_iter(k0 + 1, 1, True, True)
        return c
    lax.fori_loop(0, (NCHB - 4) // 2, pair_body, 0)

    sub_iter(NCHB - 2, 0, True, False)
    wait(g_copies(1))
    wait(s_copies(1))
    compute_pre(1)
    compute_upd(1)
    fire(s_copies(1), add=True)
    wait(s_copies(0))
    wait(s_copies(1))

    plsc.subcore_barrier()
    dbase = sid * rows_per_tile
    pltpu.sync_copy(shared_f.at[pl.ds(dbase, rows_per_tile)],
                    outf_hbm.at[pl.ds(cid * NP + dbase, rows_per_tile)])
    pltpu.sync_copy(shared_s.at[pl.ds(dbase, rows_per_tile)],
                    outs_hbm.at[pl.ds(cid * NP + dbase, rows_per_tile)])


_agg = pl.kernel(
    _agg_body,
    out_type=[jax.ShapeDtypeStruct((2 * NP, HID), jnp.float32),
              jax.ShapeDtypeStruct((2 * NP,), jnp.float32)],
    mesh=_mesh,
    scratch_types=[pltpu.VMEM_SHARED((NP, HID), jnp.float32),
                   pltpu.VMEM_SHARED((NP,), jnp.float32),
                   pltpu.VMEM((2, PB, 128), jnp.int32),
                   pltpu.VMEM((2, PB, 128), jnp.int32),
                   pltpu.VMEM((2, CHB), jnp.float32),
                   pltpu.VMEM((2, CHB), jnp.float32),
                   pltpu.VMEM((2, CHB), jnp.float32),
                   pltpu.VMEM((2, CHB, HID), jnp.float32),
                   pltpu.VMEM((2, CHB, HID), jnp.float32),
                   pltpu.VMEM((2, PB, 128), jnp.int32),
                   pltpu.VMEM((128, HID), jnp.float32),
                   pltpu.VMEM((128,), jnp.float32),
                   pltpu.SemaphoreType.DMA,
                   pltpu.SemaphoreType.DMA,
                   pltpu.SemaphoreType.DMA,
                   pltpu.SemaphoreType.DMA,
                   pltpu.SemaphoreType.DMA,
                   pltpu.SemaphoreType.DMA],
    compiler_params=_sc_params,
)


def _t1_body(x_ref, w1_ref, aw1_ref, feat_ref, s_ref, d_ref):
    hw = jnp.dot(x_ref[...], w1_ref[...], preferred_element_type=jnp.float32)
    feat_ref[...] = hw
    a = aw1_ref[...]
    s_ref[...] = jnp.dot(hw, a[:HID, :], preferred_element_type=jnp.float32)
    d_ref[...] = jnp.dot(hw, a[HID:, :], preferred_element_type=jnp.float32)


def _t1(xp, w1, aw1):
    return pl.pallas_call(
        _t1_body,
        grid=(NP // BLK,),
        in_specs=[pl.BlockSpec((BLK, IN_DIM), lambda i: (i, 0)),
                  pl.BlockSpec((IN_DIM, HID), lambda i: (0, 0)),
                  pl.BlockSpec((2 * HID, 1), lambda i: (0, 0))],
        out_specs=[pl.BlockSpec((BLK, HID), lambda i: (i, 0)),
                   pl.BlockSpec((BLK, 1), lambda i: (i, 0)),
                   pl.BlockSpec((BLK, 1), lambda i: (i, 0))],
        out_shape=[jax.ShapeDtypeStruct((NP, HID), jnp.float32),
                   jax.ShapeDtypeStruct((NP, 1), jnp.float32),
                   jax.ShapeDtypeStruct((NP, 1), jnp.float32)],
    )(xp, w1, aw1)


def _hidden_from_acc(num, ssum):
    safe = jnp.where(ssum > 0, ssum, 1.0)
    return jnp.where(ssum > 0, num / safe, 0.0)


def _t2_body(pf_ref, ps_ref, w2_ref, aw2_ref, feat_ref, s_ref, d_ref):
    num = pf_ref[0] + pf_ref[1]
    ssum = (ps_ref[0] + ps_ref[1])[:, None]
    h = _hidden_from_acc(num, ssum)
    h = jnp.where(h > 0, h, jnp.exp(jnp.minimum(h, 0.0)) - 1.0)
    feat_ref[...] = h
    w2 = w2_ref[...]
    a = aw2_ref[...]
    wa_top = jnp.dot(w2, a[:NCLS, :], preferred_element_type=jnp.float32)
    wa_bot = jnp.dot(w2, a[NCLS:, :], preferred_element_type=jnp.float32)
    s_ref[...] = jnp.dot(h, wa_top, preferred_element_type=jnp.float32)
    d_ref[...] = jnp.dot(h, wa_bot, preferred_element_type=jnp.float32)


def _t2(accf, accs, w2, aw2):
    return pl.pallas_call(
        _t2_body,
        grid=(NP // BLK,),
        in_specs=[pl.BlockSpec((2, BLK, HID), lambda i: (0, i, 0)),
                  pl.BlockSpec((2, BLK), lambda i: (0, i)),
                  pl.BlockSpec((HID, NCLS), lambda i: (0, 0)),
                  pl.BlockSpec((2 * NCLS, 1), lambda i: (0, 0))],
        out_specs=[pl.BlockSpec((BLK, HID), lambda i: (i, 0)),
                   pl.BlockSpec((BLK, 1), lambda i: (i, 0)),
                   pl.BlockSpec((BLK, 1), lambda i: (i, 0))],
        out_shape=[jax.ShapeDtypeStruct((NP, HID), jnp.float32),
                   jax.ShapeDtypeStruct((NP, 1), jnp.float32),
                   jax.ShapeDtypeStruct((NP, 1), jnp.float32)],
    )(accf, accs, w2, aw2)


def _t3_body(pf_ref, ps_ref, w2_ref, o_ref):
    num = pf_ref[0] + pf_ref[1]
    ssum = (ps_ref[0] + ps_ref[1])[:, None]
    h = _hidden_from_acc(num, ssum)
    o = jnp.dot(h, w2_ref[...], preferred_element_type=jnp.float32)
    m = jnp.max(o, axis=1, keepdims=True)
    lse = jnp.log(jnp.sum(jnp.exp(o - m), axis=1, keepdims=True)) + m
    o_ref[...] = o - lse


def _t3(accf, accs, w2):
    return pl.pallas_call(
        _t3_body,
        grid=(NP // BLK,),
        in_specs=[pl.BlockSpec((2, BLK, HID), lambda i: (0, i, 0)),
                  pl.BlockSpec((2, BLK), lambda i: (0, i)),
                  pl.BlockSpec((HID, NCLS), lambda i: (0, 0))],
        out_specs=pl.BlockSpec((BLK, NCLS), lambda i: (i, 0)),
        out_shape=jax.ShapeDtypeStruct((NP, NCLS), jnp.float32),
    )(accf, accs, w2)


def kernel(local_features, edge_index, weight1, weight2,
           att_weight1, att_weight2):
    src = edge_index[0]
    dst = edge_index[1]
    xp = jnp.pad(local_features, ((0, NP - N), (0, 0)))
    srcp = jnp.pad(src, (0, EP - E), constant_values=TRASH).reshape(EPR, 128)
    dstp = jnp.pad(dst, (0, EP - E), constant_values=TRASH).reshape(EPR, 128)

    feat1, s1, d1 = _t1(xp, weight1, att_weight1)
    att1, mparts1 = _att_max(srcp, dstp, s1.reshape(NP), d1.reshape(NP))
    m1 = _combine(mparts1)
    accf1, accs1 = _agg(srcp, dstp, att1, m1, feat1)

    feat2, s2, d2 = _t2(accf1.reshape(2, NP, HID), accs1.reshape(2, NP),
                        weight2, att_weight2)
    att2, mparts2 = _att_max(srcp, dstp, s2.reshape(NP), d2.reshape(NP))
    m2 = _combine(mparts2)
    accf2, accs2 = _agg(srcp, dstp, att2, m2, feat2)

    out = _t3(accf2.reshape(2, NP, HID), accs2.reshape(2, NP), weight2)
    return out[:N]


# speedup vs baseline: 35.3040x; 1.0161x over previous
import functools

import jax
import jax.numpy as jnp
from jax import lax
from jax.experimental import pallas as pl
from jax.experimental.pallas import tpu as pltpu
from jax.experimental.pallas import tpu_sc as plsc

N = 100000
E = 1600000
IN_DIM = 128
HID = 16
NCLS = 40

NW = 32
NP = 100352
EP = 1605632
EPR = EP // 128
CH = 1024
EW = EP // NW
NCH = EW // CH
BLK = 2048
TRASH = NP - 1

_mesh = plsc.VectorSubcoreMesh(core_axis_name="c", subcore_axis_name="s")
_sc_params = pltpu.CompilerParams(needs_layout_passes=False,
                                  use_tc_tiling_on_sc=False)


def _worker_id():
    return lax.axis_index("s") * 2 + lax.axis_index("c")


CHA = 512
NCHA = EW // CHA
PA = CHA // 128


def _att_max_body(src_hbm, dst_hbm, s_hbm, d_hbm, att_hbm, mparts_hbm,
                  m_loc, src_v, dst_v, sg_v, dg_v, att_v, sidx_v,
                  sem_i0, sem_i1, sem_g0, sem_g1, sem_s0, sem_s1):
    wid = _worker_id()
    sem_i = (sem_i0, sem_i1)
    sem_g = (sem_g0, sem_g1)
    sem_s = (sem_s0, sem_s1)

    def init_body(i, c):
        m_loc[pl.ds(i * 16, 16)] = jnp.full((16,), -1e30, jnp.float32)
        return c
    lax.fori_loop(0, NP // 16, init_body, 0)

    def idx_copies(cj, b):
        rb = wid * (EW // 128) + cj * PA
        return [pltpu.make_async_copy(src_hbm.at[pl.ds(rb, PA)],
                                      src_v.at[b], sem_i[b]),
                pltpu.make_async_copy(dst_hbm.at[pl.ds(rb, PA)],
                                      dst_v.at[b], sem_i[b])]

    def g_copies(b):
        out = []
        for j in range(PA):
            out.append(pltpu.make_async_copy(
                s_hbm.at[src_v.at[b, j]],
                sg_v.at[b, pl.ds(j * 128, 128)], sem_g[b]))
            out.append(pltpu.make_async_copy(
                d_hbm.at[dst_v.at[b, j]],
                dg_v.at[b, pl.ds(j * 128, 128)], sem_g[b]))
        return out

    def s_copies(cj, b):
        base = wid * EW + cj * CHA
        return [pltpu.make_async_copy(att_v.at[b],
                                      att_hbm.at[pl.ds(base, CHA)],
                                      sem_s[b])]

    def fire(copies):
        for cp in copies:
            cp.start()

    def wait(copies):
        for cp in copies:
            cp.wait()

    def compute_pre(b):
        def att_body(vi, cc):
            sl = pl.ds(vi * 16, 16)
            a = sg_v[b, sl] + dg_v[b, sl]
            a = jnp.maximum(a, a * 0.01)
            att_v[b, sl] = a
            sidx_v[b, sl] = src_v[b, vi // 8, pl.ds((vi % 8) * 16, 16)]
            return cc
        lax.fori_loop(0, CHA // 16, att_body, 0)

    def compute_rmw(b):
        def vec_body(vi, nbad):
            sl = pl.ds(vi * 16, 16)
            a = att_v[b, sl]
            idx = sidx_v[b, sl]
            cur = plsc.load_gather(m_loc, (idx,))
            plsc.store_scatter(m_loc, (idx,), jnp.maximum(cur, a))
            chk = plsc.load_gather(m_loc, (idx,))
            bad = a > chk
            return nbad + jnp.sum(bad.astype(jnp.int32))
        nbad = lax.fori_loop(0, CHA // 16, vec_body, jnp.int32(0))

        def fix_cond(st):
            return st > 0

        def fix_body(st):
            def fb(vi, nb):
                sl = pl.ds(vi * 16, 16)
                a = att_v[b, sl]
                idx = sidx_v[b, sl]
                cur = plsc.load_gather(m_loc, (idx,))
                bad = a > cur
                plsc.store_scatter(m_loc, (idx,), jnp.maximum(cur, a),
                                   mask=bad)
                chk = plsc.load_gather(m_loc, (idx,))
                bad2 = a > chk
                return nb + jnp.sum(bad2.astype(jnp.int32))
            return lax.fori_loop(0, CHA // 16, fb, jnp.int32(0))
        lax.while_loop(fix_cond, fix_body, nbad)

    def sub_iter(k, b, drain_s, fire_next_idx):
        wait(idx_copies(k + 1, 1 - b))
        fire(g_copies(1 - b))
        wait(g_copies(b))
        if drain_s:
            wait(s_copies(k - 2, b))
        compute_pre(b)
        if fire_next_idx:
            fire(idx_copies(k + 2, b))
        compute_rmw(b)
        fire(s_copies(k, b))

    fire(idx_copies(0, 0))
    fire(idx_copies(1, 1))
    wait(idx_copies(0, 0))
    fire(g_copies(0))
    sub_iter(0, 0, False, True)
    sub_iter(1, 1, False, True)

    def pair_body(g, c):
        k0 = 2 + g * 2
        sub_iter(k0, 0, True, True)
        sub_iter(k0 + 1, 1, True, True)
        return c
    lax.fori_loop(0, (NCHA - 4) // 2, pair_body, 0)

    sub_iter(NCHA - 2, 0, True, False)
    wait(g_copies(1))
    wait(s_copies(NCHA - 3, 1))
    compute_pre(1)
    compute_rmw(1)
    fire(s_copies(NCHA - 1, 1))
    wait(s_copies(NCHA - 2, 0))
    wait(s_copies(NCHA - 1, 1))

    pltpu.sync_copy(m_loc, mparts_hbm.at[pl.ds(wid * NP, NP)])


_att_max = pl.kernel(
    _att_max_body,
    out_type=[jax.ShapeDtypeStruct((EP,), jnp.float32),
              jax.ShapeDtypeStruct((NW * NP,), jnp.float32)],
    mesh=_mesh,
    scratch_types=[pltpu.VMEM((NP,), jnp.float32),
                   pltpu.VMEM((2, PA, 128), jnp.int32),
                   pltpu.VMEM((2, PA, 128), jnp.int32),
                   pltpu.VMEM((2, CHA), jnp.float32),
                   pltpu.VMEM((2, CHA), jnp.float32),
                   pltpu.VMEM((2, CHA), jnp.float32),
                   pltpu.VMEM((2, CHA), jnp.int32),
                   pltpu.SemaphoreType.DMA,
                   pltpu.SemaphoreType.DMA,
                   pltpu.SemaphoreType.DMA,
                   pltpu.SemaphoreType.DMA,
                   pltpu.SemaphoreType.DMA,
                   pltpu.SemaphoreType.DMA],
    compiler_params=_sc_params,
)


def _combine_body(mparts_hbm, m_hbm, acc_v, tmp_v):
    wid = _worker_id()
    sl_n = NP // NW
    base = wid * sl_n
    pltpu.sync_copy(mparts_hbm.at[pl.ds(base, sl_n)], acc_v)

    def part_body(t, c):
        pltpu.sync_copy(mparts_hbm.at[pl.ds(t * NP + base, sl_n)], tmp_v)

        def vec_body(i, cc):
            sl = pl.ds(i * 16, 16)
            acc_v[sl] = jnp.maximum(acc_v[sl], tmp_v[sl])
            return cc
        lax.fori_loop(0, sl_n // 16, vec_body, 0)
        return c
    lax.fori_loop(1, NW, part_body, 0)
    pltpu.sync_copy(acc_v, m_hbm.at[pl.ds(base, sl_n)])


_combine = pl.kernel(
    _combine_body,
    out_type=jax.ShapeDtypeStruct((NP,), jnp.float32),
    mesh=_mesh,
    scratch_types=[pltpu.VMEM((NP // NW,), jnp.float32),
                   pltpu.VMEM((NP // NW,), jnp.float32)],
    compiler_params=_sc_params,
)


CHB = 256
NCHB = EW // CHB
PB = CHB // 128


def _agg_body(src_hbm, dst_hbm, att_hbm, m_hbm, feat_hbm,
              outf_hbm, outs_hbm,
              shared_f, shared_s, src_v, dst_v, att_v, mg_v, e_v,
              feat_v, updf_v, sidx_v, zf_v, zs_v,
              sem_i0, sem_i1, sem_g0, sem_g1, sem_s0, sem_s1):
    cid = lax.axis_index("c")
    sid = lax.axis_index("s")
    wid = sid * 2 + cid
    rows_per_tile = NP // 16
    lanes = lax.iota(jnp.int32, 16)
    sem_i = (sem_i0, sem_i1)
    sem_g = (sem_g0, sem_g1)
    sem_s = (sem_s0, sem_s1)

    z16 = jnp.zeros((16,), jnp.float32)

    def zf_body(r, c):
        zf_v[r, pl.ds(0, HID)] = z16
        return c
    lax.fori_loop(0, 128, zf_body, 0)

    def zs_body(i, c):
        zs_v[pl.ds(i * 16, 16)] = z16
        return c
    lax.fori_loop(0, 8, zs_body, 0)
    zbase = sid * rows_per_tile

    def zero_body(i, c):
        pltpu.sync_copy(zf_v, shared_f.at[pl.ds(zbase + i * 128, 128)])
        pltpu.sync_copy(zs_v, shared_s.at[pl.ds(zbase + i * 128, 128)])
        return c
    lax.fori_loop(0, rows_per_tile // 128, zero_body, 0)
    plsc.subcore_barrier()

    def idx_copies(cj, b):
        rb = wid * (EW // 128) + cj * PB
        base = wid * EW + cj * CHB
        return [pltpu.make_async_copy(src_hbm.at[pl.ds(rb, PB)],
                                      src_v.at[b], sem_i[b]),
                pltpu.make_async_copy(dst_hbm.at[pl.ds(rb, PB)],
                                      dst_v.at[b], sem_i[b]),
                pltpu.make_async_copy(att_hbm.at[pl.ds(base, CHB)],
                                      att_v.at[b], sem_i[b])]

    def g_copies(b):
        out = []
        for j in range(PB):
            out.append(pltpu.make_async_copy(
                m_hbm.at[src_v.at[b, j]],
                mg_v.at[b, pl.ds(j * 128, 128)], sem_g[b]))
            out.append(pltpu.make_async_copy(
                feat_hbm.at[dst_v.at[b, j]],
                feat_v.at[b, pl.ds(j * 128, 128)], sem_g[b]))
        return out

    def s_copies(b):
        out = []
        for j in range(PB):
            out.append(pltpu.make_async_copy(
                updf_v.at[b, pl.ds(j * 128, 128)],
                shared_f.at[sidx_v.at[b, j]], sem_s[b]))
            out.append(pltpu.make_async_copy(
                e_v.at[b, pl.ds(j * 128, 128)],
                shared_s.at[sidx_v.at[b, j]], sem_s[b]))
        return out

    def fire(copies, add=False):
        for cp in copies:
            cp.start(add=add)

    def wait(copies):
        for cp in copies:
            cp.wait()

    def compute_pre(b):
        def exp_body(vi, cc):
            sl = pl.ds(vi * 16, 16)
            e_v[b, sl] = jnp.exp(att_v[b, sl] - mg_v[b, sl])
            return cc
        lax.fori_loop(0, CHB // 16, exp_body, 0)

        def cpidx_body(vi, cc):
            sl = pl.ds(vi * 16, 16)
            for j in range(PB):
                sidx_v[b, j, sl] = src_v[b, j, sl]
            return cc
        lax.fori_loop(0, 8, cpidx_body, 0)

    def compute_upd(b):
        fv = feat_v.at[b]
        uv = updf_v.at[b]

        def upd_body(vi, cc):
            rows = lanes + vi * 16
            e16 = e_v[b, pl.ds(vi * 16, 16)]
            for col in range(HID):
                cc16 = jnp.full((16,), col, jnp.int32)
                f = plsc.load_gather(fv, (rows, cc16))
                plsc.store_scatter(uv, (rows, cc16), f * e16)
            return cc
        lax.fori_loop(0, CHB // 16, upd_body, 0)

    def sub_iter(k, b, drain_s, fire_next_idx):
        wait(idx_copies(k + 1, 1 - b))
        fire(g_copies(1 - b))
        wait(g_copies(b))
        if drain_s:
            wait(s_copies(b))
        compute_pre(b)
        if fire_next_idx:
            fire(idx_copies(k + 2, b))
        compute_upd(b)
        fire(s_copies(b), add=True)

    fire(idx_copies(0, 0))
    fire(idx_copies(1, 1))
    wait(idx_copies(0, 0))
    fire(g_copies(0))
    sub_iter(0, 0, False, True)
    sub_iter(1, 1, False, True)

    def pair_body(g, c):
        k0 = 2 + g * 2
        sub_iter(k0, 0, True, True)
        sub_iter(k0 + 1, 1, True, True)
        return c
    lax.fori_loop(0, (NCHB - 4) // 2, pair_body, 0)

    sub_iter(NCHB - 2, 0, True, False)
    wait(g_copies(1))
    wait(s_copies(1))
    compute_pre(1)
    compute_upd(1)
    fire(s_copies(1), add=True)
    wait(s_copies(0))
    wait(s_copies(1))

    plsc.subcore_barrier()
    dbase = sid * rows_per_tile
    pltpu.sync_copy(shared_f.at[pl.ds(dbase, rows_per_tile)],
                    outf_hbm.at[pl.ds(cid * NP + dbase, rows_per_tile)])
    pltpu.sync_copy(shared_s.at[pl.ds(dbase, rows_per_tile)],
                    outs_hbm.at[pl.ds(cid * NP + dbase, rows_per_tile)])


_agg = pl.kernel(
    _agg_body,
    out_type=[jax.ShapeDtypeStruct((2 * NP, HID), jnp.float32),
              jax.ShapeDtypeStruct((2 * NP,), jnp.float32)],
    mesh=_mesh,
    scratch_types=[pltpu.VMEM_SHARED((NP, HID), jnp.float32),
                   pltpu.VMEM_SHARED((NP,), jnp.float32),
                   pltpu.VMEM((2, PB, 128), jnp.int32),
                   pltpu.VMEM((2, PB, 128), jnp.int32),
                   pltpu.VMEM((2, CHB), jnp.float32),
                   pltpu.VMEM((2, CHB), jnp.float32),
                   pltpu.VMEM((2, CHB), jnp.float32),
                   pltpu.VMEM((2, CHB, HID), jnp.float32),
                   pltpu.VMEM((2, CHB, HID), jnp.float32),
                   pltpu.VMEM((2, PB, 128), jnp.int32),
                   pltpu.VMEM((128, HID), jnp.float32),
                   pltpu.VMEM((128,), jnp.float32),
                   pltpu.SemaphoreType.DMA,
                   pltpu.SemaphoreType.DMA,
                   pltpu.SemaphoreType.DMA,
                   pltpu.SemaphoreType.DMA,
                   pltpu.SemaphoreType.DMA,
                   pltpu.SemaphoreType.DMA],
    compiler_params=_sc_params,
)


def _t1_body(x_ref, w1_ref, aw1_ref, feat_ref, s_ref, d_ref):
    hw = jnp.dot(x_ref[...], w1_ref[...], preferred_element_type=jnp.float32)
    feat_ref[...] = hw
    a = aw1_ref[...]
    s_ref[...] = jnp.dot(hw, a[:HID, :], preferred_element_type=jnp.float32)
    d_ref[...] = jnp.dot(hw, a[HID:, :], preferred_element_type=jnp.float32)


def _t1(xp, w1, aw1):
    return pl.pallas_call(
        _t1_body,
        grid=(NP // BLK,),
        in_specs=[pl.BlockSpec((BLK, IN_DIM), lambda i: (i, 0)),
                  pl.BlockSpec((IN_DIM, HID), lambda i: (0, 0)),
                  pl.BlockSpec((2 * HID, 1), lambda i: (0, 0))],
        out_specs=[pl.BlockSpec((BLK, HID), lambda i: (i, 0)),
                   pl.BlockSpec((BLK, 1), lambda i: (i, 0)),
                   pl.BlockSpec((BLK, 1), lambda i: (i, 0))],
        out_shape=[jax.ShapeDtypeStruct((NP, HID), jnp.float32),
                   jax.ShapeDtypeStruct((NP, 1), jnp.float32),
                   jax.ShapeDtypeStruct((NP, 1), jnp.float32)],
    )(xp, w1, aw1)


def _hidden_from_acc(num, ssum):
    safe = jnp.where(ssum > 0, ssum, 1.0)
    return jnp.where(ssum > 0, num / safe, 0.0)


def _t2_body(pf_ref, ps_ref, w2_ref, aw2_ref, feat_ref, s_ref, d_ref):
    num = pf_ref[0] + pf_ref[1]
    ssum = (ps_ref[0] + ps_ref[1])[:, None]
    h = _hidden_from_acc(num, ssum)
    h = jnp.where(h > 0, h, jnp.exp(jnp.minimum(h, 0.0)) - 1.0)
    feat_ref[...] = h
    w2 = w2_ref[...]
    a = aw2_ref[...]
    wa_top = jnp.dot(w2, a[:NCLS, :], preferred_element_type=jnp.float32)
    wa_bot = jnp.dot(w2, a[NCLS:, :], preferred_element_type=jnp.float32)
    s_ref[...] = jnp.dot(h, wa_top, preferred_element_type=jnp.float32)
    d_ref[...] = jnp.dot(h, wa_bot, preferred_element_type=jnp.float32)


def _t2(accf, accs, w2, aw2):
    return pl.pallas_call(
        _t2_body,
        grid=(NP // BLK,),
        in_specs=[pl.BlockSpec((2, BLK, HID), lambda i: (0, i, 0)),
                  pl.BlockSpec((2, BLK), lambda i: (0, i)),
                  pl.BlockSpec((HID, NCLS), lambda i: (0, 0)),
                  pl.BlockSpec((2 * NCLS, 1), lambda i: (0, 0))],
        out_specs=[pl.BlockSpec((BLK, HID), lambda i: (i, 0)),
                   pl.BlockSpec((BLK, 1), lambda i: (i, 0)),
                   pl.BlockSpec((BLK, 1), lambda i: (i, 0))],
        out_shape=[jax.ShapeDtypeStruct((NP, HID), jnp.float32),
                   jax.ShapeDtypeStruct((NP, 1), jnp.float32),
                   jax.ShapeDtypeStruct((NP, 1), jnp.float32)],
    )(accf, accs, w2, aw2)


def _t3_body(pf_ref, ps_ref, w2_ref, o_ref):
    num = pf_ref[0] + pf_ref[1]
    ssum = (ps_ref[0] + ps_ref[1])[:, None]
    h = _hidden_from_acc(num, ssum)
    o = jnp.dot(h, w2_ref[...], preferred_element_type=jnp.float32)
    m = jnp.max(o, axis=1, keepdims=True)
    lse = jnp.log(jnp.sum(jnp.exp(o - m), axis=1, keepdims=True)) + m
    o_ref[...] = o - lse


def _t3(accf, accs, w2):
    return pl.pallas_call(
        _t3_body,
        grid=(NP // BLK,),
        in_specs=[pl.BlockSpec((2, BLK, HID), lambda i: (0, i, 0)),
                  pl.BlockSpec((2, BLK), lambda i: (0, i)),
                  pl.BlockSpec((HID, NCLS), lambda i: (0, 0))],
        out_specs=pl.BlockSpec((BLK, NCLS), lambda i: (i, 0)),
        out_shape=jax.ShapeDtypeStruct((NP, NCLS), jnp.float32),
    )(accf, accs, w2)


def kernel(local_features, edge_index, weight1, weight2,
           att_weight1, att_weight2):
    src = edge_index[0]
    dst = edge_index[1]
    xp = jnp.pad(local_features, ((0, NP - N), (0, 0)))
    srcp = jnp.pad(src, (0, EP - E), constant_values=TRASH).reshape(EPR, 128)
    dstp = jnp.pad(dst, (0, EP - E), constant_values=TRASH).reshape(EPR, 128)

    feat1, s1, d1 = _t1(xp, weight1, att_weight1)
    att1, mparts1 = _att_max(srcp, dstp, s1.reshape(NP), d1.reshape(NP))
    m1 = _combine(mparts1)
    accf1, accs1 = _agg(srcp, dstp, att1, m1, feat1)

    feat2, s2, d2 = _t2(accf1.reshape(2, NP, HID), accs1.reshape(2, NP),
                        weight2, att_weight2)
    att2, mparts2 = _att_max(srcp, dstp, s2.reshape(NP), d2.reshape(NP))
    m2 = _combine(mparts2)
    accf2, accs2 = _agg(srcp, dstp, att2, m2, feat2)

    out = _t3(accf2.reshape(2, NP, HID), accs2.reshape(2, NP), weight2)
    return out[:N]


# speedup vs baseline: 35.5109x; 1.0059x over previous
import functools

import jax
import jax.numpy as jnp
from jax import lax
from jax.experimental import pallas as pl
from jax.experimental.pallas import tpu as pltpu
from jax.experimental.pallas import tpu_sc as plsc

N = 100000
E = 1600000
IN_DIM = 128
HID = 16
NCLS = 40

NW = 32
NP = 100352
EP = 1605632
EPR = EP // 128
CH = 1024
EW = EP // NW
NCH = EW // CH
BLK = 2048
TRASH = NP - 1

_mesh = plsc.VectorSubcoreMesh(core_axis_name="c", subcore_axis_name="s")
_sc_params = pltpu.CompilerParams(needs_layout_passes=False,
                                  use_tc_tiling_on_sc=False)


def _worker_id():
    return lax.axis_index("s") * 2 + lax.axis_index("c")


CHA = 512
NCHA = EW // CHA
PA = CHA // 128


def _att_max_body(src_hbm, dst_hbm, s_hbm, d_hbm, att_hbm, mparts_hbm,
                  m_loc, src_v, dst_v, sg_v, dg_v, att_v, sidx_v,
                  sem_i0, sem_i1, sem_g0, sem_g1, sem_s0, sem_s1):
    wid = _worker_id()
    sem_i = (sem_i0, sem_i1)
    sem_g = (sem_g0, sem_g1)
    sem_s = (sem_s0, sem_s1)

    def init_body(i, c):
        m_loc[pl.ds(i * 16, 16)] = jnp.full((16,), -1e30, jnp.float32)
        return c
    lax.fori_loop(0, NP // 16, init_body, 0)

    def idx_copies(cj, b):
        rb = wid * (EW // 128) + cj * PA
        return [pltpu.make_async_copy(src_hbm.at[pl.ds(rb, PA)],
                                      src_v.at[b], sem_i[b]),
                pltpu.make_async_copy(dst_hbm.at[pl.ds(rb, PA)],
                                      dst_v.at[b], sem_i[b])]

    def g_copies(b):
        out = []
        for j in range(PA):
            out.append(pltpu.make_async_copy(
                s_hbm.at[src_v.at[b, j]],
                sg_v.at[b, pl.ds(j * 128, 128)], sem_g[b]))
            out.append(pltpu.make_async_copy(
                d_hbm.at[dst_v.at[b, j]],
                dg_v.at[b, pl.ds(j * 128, 128)], sem_g[b]))
        return out

    def s_copies(cj, b):
        base = wid * EW + cj * CHA
        return [pltpu.make_async_copy(att_v.at[b],
                                      att_hbm.at[pl.ds(base, CHA)],
                                      sem_s[b])]

    def fire(copies):
        for cp in copies:
            cp.start()

    def wait(copies):
        for cp in copies:
            cp.wait()

    def compute_pre(b):
        def att_body(vi, cc):
            sl = pl.ds(vi * 16, 16)
            a = sg_v[b, sl] + dg_v[b, sl]
            a = jnp.maximum(a, a * 0.01)
            att_v[b, sl] = a
            sidx_v[b, sl] = src_v[b, vi // 8, pl.ds((vi % 8) * 16, 16)]
            return cc
        lax.fori_loop(0, CHA // 16, att_body, 0)

    def compute_rmw(b):
        def vec_body(vi, nbad):
            sl = pl.ds(vi * 16, 16)
            a = att_v[b, sl]
            idx = sidx_v[b, sl]
            cur = plsc.load_gather(m_loc, (idx,))
            plsc.store_scatter(m_loc, (idx,), jnp.maximum(cur, a))
            chk = plsc.load_gather(m_loc, (idx,))
            bad = a > chk
            return nbad + jnp.sum(bad.astype(jnp.int32))
        nbad = lax.fori_loop(0, CHA // 16, vec_body, jnp.int32(0))

        def fix_cond(st):
            return st > 0

        def fix_body(st):
            def fb(vi, nb):
                sl = pl.ds(vi * 16, 16)
                a = att_v[b, sl]
                idx = sidx_v[b, sl]
                cur = plsc.load_gather(m_loc, (idx,))
                bad = a > cur
                plsc.store_scatter(m_loc, (idx,), jnp.maximum(cur, a),
                                   mask=bad)
                chk = plsc.load_gather(m_loc, (idx,))
                bad2 = a > chk
                return nb + jnp.sum(bad2.astype(jnp.int32))
            return lax.fori_loop(0, CHA // 16, fb, jnp.int32(0))
        lax.while_loop(fix_cond, fix_body, nbad)

    def sub_iter(k, b, drain_s, fire_next_idx):
        wait(idx_copies(k + 1, 1 - b))
        fire(g_copies(1 - b))
        wait(g_copies(b))
        if drain_s:
            wait(s_copies(k - 2, b))
        compute_pre(b)
        if fire_next_idx:
            fire(idx_copies(k + 2, b))
        compute_rmw(b)
        fire(s_copies(k, b))

    fire(idx_copies(0, 0))
    fire(idx_copies(1, 1))
    wait(idx_copies(0, 0))
    fire(g_copies(0))
    sub_iter(0, 0, False, True)
    sub_iter(1, 1, False, True)

    def pair_body(g, c):
        k0 = 2 + g * 2
        sub_iter(k0, 0, True, True)
        sub_iter(k0 + 1, 1, True, True)
        return c
    lax.fori_loop(0, (NCHA - 4) // 2, pair_body, 0)

    sub_iter(NCHA - 2, 0, True, False)
    wait(g_copies(1))
    wait(s_copies(NCHA - 3, 1))
    compute_pre(1)
    compute_rmw(1)
    fire(s_copies(NCHA - 1, 1))
    wait(s_copies(NCHA - 2, 0))
    wait(s_copies(NCHA - 1, 1))

    pltpu.sync_copy(m_loc, mparts_hbm.at[pl.ds(wid * NP, NP)])


_att_max = pl.kernel(
    _att_max_body,
    out_type=[jax.ShapeDtypeStruct((EP,), jnp.float32),
              jax.ShapeDtypeStruct((NW * NP,), jnp.float32)],
    mesh=_mesh,
    scratch_types=[pltpu.VMEM((NP,), jnp.float32),
                   pltpu.VMEM((2, PA, 128), jnp.int32),
                   pltpu.VMEM((2, PA, 128), jnp.int32),
                   pltpu.VMEM((2, CHA), jnp.float32),
                   pltpu.VMEM((2, CHA), jnp.float32),
                   pltpu.VMEM((2, CHA), jnp.float32),
                   pltpu.VMEM((2, CHA), jnp.int32),
                   pltpu.SemaphoreType.DMA,
                   pltpu.SemaphoreType.DMA,
                   pltpu.SemaphoreType.DMA,
                   pltpu.SemaphoreType.DMA,
                   pltpu.SemaphoreType.DMA,
                   pltpu.SemaphoreType.DMA],
    compiler_params=_sc_params,
)


def _combine_body(mparts_hbm, m_hbm, acc_v, tmp_v):
    wid = _worker_id()
    sl_n = NP // NW
    base = wid * sl_n
    pltpu.sync_copy(mparts_hbm.at[pl.ds(base, sl_n)], acc_v)

    def part_body(t, c):
        pltpu.sync_copy(mparts_hbm.at[pl.ds(t * NP + base, sl_n)], tmp_v)

        def vec_body(i, cc):
            sl = pl.ds(i * 16, 16)
            acc_v[sl] = jnp.maximum(acc_v[sl], tmp_v[sl])
            return cc
        lax.fori_loop(0, sl_n // 16, vec_body, 0)
        return c
    lax.fori_loop(1, NW, part_body, 0)
    pltpu.sync_copy(acc_v, m_hbm.at[pl.ds(base, sl_n)])


_combine = pl.kernel(
    _combine_body,
    out_type=jax.ShapeDtypeStruct((NP,), jnp.float32),
    mesh=_mesh,
    scratch_types=[pltpu.VMEM((NP // NW,), jnp.float32),
                   pltpu.VMEM((NP // NW,), jnp.float32)],
    compiler_params=_sc_params,
)


CHB = 256
NCHB = EW // CHB
PB = CHB // 128


def _agg_body(src_hbm, dst_hbm, att_hbm, m_hbm, feat_hbm,
              outf_hbm, outs_hbm,
              shared_f, shared_s, src_v, dst_v, att_v, mg_v, e_v,
              feat_v, updf_v, sidx_v, zf_v, zs_v,
              sem_i0, sem_i1, sem_g0, sem_g1, sem_s0, sem_s1):
    cid = lax.axis_index("c")
    sid = lax.axis_index("s")
    wid = sid * 2 + cid
    rows_per_tile = NP // 16
    lanes = lax.iota(jnp.int32, 16)
    sem_i = (sem_i0, sem_i1)
    sem_g = (sem_g0, sem_g1)
    sem_s = (sem_s0, sem_s1)

    z16 = jnp.zeros((16,), jnp.float32)

    def zf_body(r, c):
        zf_v[r, pl.ds(0, HID)] = z16
        return c
    lax.fori_loop(0, 128, zf_body, 0)

    def zs_body(i, c):
        zs_v[pl.ds(i * 16, 16)] = z16
        return c
    lax.fori_loop(0, 8, zs_body, 0)
    zbase = sid * rows_per_tile

    def zero_fire(i, c):
        pltpu.async_copy(zf_v, shared_f.at[pl.ds(zbase + i * 128, 128)],
                         sem_s[0])
        pltpu.async_copy(zs_v, shared_s.at[pl.ds(zbase + i * 128, 128)],
                         sem_s[1])
        return c
    lax.fori_loop(0, rows_per_tile // 128, zero_fire, 0)

    def zero_drain(i, c):
        pltpu.make_async_copy(zf_v, shared_f.at[pl.ds(zbase + i * 128, 128)],
                              sem_s[0]).wait()
        pltpu.make_async_copy(zs_v, shared_s.at[pl.ds(zbase + i * 128, 128)],
                              sem_s[1]).wait()
        return c
    lax.fori_loop(0, rows_per_tile // 128, zero_drain, 0)
    plsc.subcore_barrier()

    def idx_copies(cj, b):
        rb = wid * (EW // 128) + cj * PB
        base = wid * EW + cj * CHB
        return [pltpu.make_async_copy(src_hbm.at[pl.ds(rb, PB)],
                                      src_v.at[b], sem_i[b]),
                pltpu.make_async_copy(dst_hbm.at[pl.ds(rb, PB)],
                                      dst_v.at[b], sem_i[b]),
                pltpu.make_async_copy(att_hbm.at[pl.ds(base, CHB)],
                                      att_v.at[b], sem_i[b])]

    def g_copies(b):
        out = []
        for j in range(PB):
            out.append(pltpu.make_async_copy(
                m_hbm.at[src_v.at[b, j]],
                mg_v.at[b, pl.ds(j * 128, 128)], sem_g[b]))
            out.append(pltpu.make_async_copy(
                feat_hbm.at[dst_v.at[b, j]],
                feat_v.at[b, pl.ds(j * 128, 128)], sem_g[b]))
        return out

    def s_copies(b):
        out = []
        for j in range(PB):
            out.append(pltpu.make_async_copy(
                updf_v.at[b, pl.ds(j * 128, 128)],
                shared_f.at[sidx_v.at[b, j]], sem_s[b]))
            out.append(pltpu.make_async_copy(
                e_v.at[b, pl.ds(j * 128, 128)],
                shared_s.at[sidx_v.at[b, j]], sem_s[b]))
        return out

    def fire(copies, add=False):
        for cp in copies:
            cp.start(add=add)

    def wait(copies):
        for cp in copies:
            cp.wait()

    def compute_pre(b):
        def exp_body(vi, cc):
            sl = pl.ds(vi * 16, 16)
            e_v[b, sl] = jnp.exp(att_v[b, sl] - mg_v[b, sl])
            return cc
        lax.fori_loop(0, CHB // 16, exp_body, 0)

        def cpidx_body(vi, cc):
            sl = pl.ds(vi * 16, 16)
            for j in range(PB):
                sidx_v[b, j, sl] = src_v[b, j, sl]
            return cc
        lax.fori_loop(0, 8, cpidx_body, 0)

    def compute_upd(b):
        fv = feat_v.at[b]
        uv = updf_v.at[b]

        def upd_body(vi, cc):
            rows = lanes + vi * 16
            e16 = e_v[b, pl.ds(vi * 16, 16)]
            for col in range(HID):
                cc16 = jnp.full((16,), col, jnp.int32)
                f = plsc.load_gather(fv, (rows, cc16))
                plsc.store_scatter(uv, (rows, cc16), f * e16)
            return cc
        lax.fori_loop(0, CHB // 16, upd_body, 0)

    def sub_iter(k, b, drain_s, fire_next_idx):
        wait(idx_copies(k + 1, 1 - b))
        fire(g_copies(1 - b))
        wait(g_copies(b))
        if drain_s:
            wait(s_copies(b))
        compute_pre(b)
        if fire_next_idx:
            fire(idx_copies(k + 2, b))
        compute_upd(b)
        fire(s_copies(b), add=True)

    fire(idx_copies(0, 0))
    fire(idx_copies(1, 1))
    wait(idx_copies(0, 0))
    fire(g_copies(0))
    sub_iter(0, 0, False, True)
    sub_iter(1, 1, False, True)

    def pair_body(g, c):
        k0 = 2 + g * 2
        sub_iter(k0, 0, True, True)
        sub_iter(k0 + 1, 1, True, True)
        return c
    lax.fori_loop(0, (NCHB - 4) // 2, pair_body, 0)

    sub_iter(NCHB - 2, 0, True, False)
    wait(g_copies(1))
    wait(s_copies(1))
    compute_pre(1)
    compute_upd(1)
    fire(s_copies(1), add=True)
    wait(s_copies(0))
    wait(s_copies(1))

    plsc.subcore_barrier()
    dbase = sid * rows_per_tile
    pltpu.sync_copy(shared_f.at[pl.ds(dbase, rows_per_tile)],
                    outf_hbm.at[pl.ds(cid * NP + dbase, rows_per_tile)])
    pltpu.sync_copy(shared_s.at[pl.ds(dbase, rows_per_tile)],
                    outs_hbm.at[pl.ds(cid * NP + dbase, rows_per_tile)])


_agg = pl.kernel(
    _agg_body,
    out_type=[jax.ShapeDtypeStruct((2 * NP, HID), jnp.float32),
              jax.ShapeDtypeStruct((2 * NP,), jnp.float32)],
    mesh=_mesh,
    scratch_types=[pltpu.VMEM_SHARED((NP, HID), jnp.float32),
                   pltpu.VMEM_SHARED((NP,), jnp.float32),
                   pltpu.VMEM((2, PB, 128), jnp.int32),
                   pltpu.VMEM((2, PB, 128), jnp.int32),
                   pltpu.VMEM((2, CHB), jnp.float32),
                   pltpu.VMEM((2, CHB), jnp.float32),
                   pltpu.VMEM((2, CHB), jnp.float32),
                   pltpu.VMEM((2, CHB, HID), jnp.float32),
                   pltpu.VMEM((2, CHB, HID), jnp.float32),
                   pltpu.VMEM((2, PB, 128), jnp.int32),
                   pltpu.VMEM((128, HID), jnp.float32),
                   pltpu.VMEM((128,), jnp.float32),
                   pltpu.SemaphoreType.DMA,
                   pltpu.SemaphoreType.DMA,
                   pltpu.SemaphoreType.DMA,
                   pltpu.SemaphoreType.DMA,
                   pltpu.SemaphoreType.DMA,
                   pltpu.SemaphoreType.DMA],
    compiler_params=_sc_params,
)


def _t1_body(x_ref, w1_ref, aw1_ref, feat_ref, s_ref, d_ref):
    hw = jnp.dot(x_ref[...], w1_ref[...], preferred_element_type=jnp.float32)
    feat_ref[...] = hw
    a = aw1_ref[...]
    s_ref[...] = jnp.dot(hw, a[:HID, :], preferred_element_type=jnp.float32)
    d_ref[...] = jnp.dot(hw, a[HID:, :], preferred_element_type=jnp.float32)


def _t1(xp, w1, aw1):
    return pl.pallas_call(
        _t1_body,
        grid=(NP // BLK,),
        in_specs=[pl.BlockSpec((BLK, IN_DIM), lambda i: (i, 0)),
                  pl.BlockSpec((IN_DIM, HID), lambda i: (0, 0)),
                  pl.BlockSpec((2 * HID, 1), lambda i: (0, 0))],
        out_specs=[pl.BlockSpec((BLK, HID), lambda i: (i, 0)),
                   pl.BlockSpec((BLK, 1), lambda i: (i, 0)),
                   pl.BlockSpec((BLK, 1), lambda i: (i, 0))],
        out_shape=[jax.ShapeDtypeStruct((NP, HID), jnp.float32),
                   jax.ShapeDtypeStruct((NP, 1), jnp.float32),
                   jax.ShapeDtypeStruct((NP, 1), jnp.float32)],
    )(xp, w1, aw1)


def _hidden_from_acc(num, ssum):
    safe = jnp.where(ssum > 0, ssum, 1.0)
    return jnp.where(ssum > 0, num / safe, 0.0)


def _t2_body(pf_ref, ps_ref, w2_ref, aw2_ref, feat_ref, s_ref, d_ref):
    num = pf_ref[0] + pf_ref[1]
    ssum = (ps_ref[0] + ps_ref[1])[:, None]
    h = _hidden_from_acc(num, ssum)
    h = jnp.where(h > 0, h, jnp.exp(jnp.minimum(h, 0.0)) - 1.0)
    feat_ref[...] = h
    w2 = w2_ref[...]
    a = aw2_ref[...]
    wa_top = jnp.dot(w2, a[:NCLS, :], preferred_element_type=jnp.float32)
    wa_bot = jnp.dot(w2, a[NCLS:, :], preferred_element_type=jnp.float32)
    s_ref[...] = jnp.dot(h, wa_top, preferred_element_type=jnp.float32)
    d_ref[...] = jnp.dot(h, wa_bot, preferred_element_type=jnp.float32)


def _t2(accf, accs, w2, aw2):
    return pl.pallas_call(
        _t2_body,
        grid=(NP // BLK,),
        in_specs=[pl.BlockSpec((2, BLK, HID), lambda i: (0, i, 0)),
                  pl.BlockSpec((2, BLK), lambda i: (0, i)),
                  pl.BlockSpec((HID, NCLS), lambda i: (0, 0)),
                  pl.BlockSpec((2 * NCLS, 1), lambda i: (0, 0))],
        out_specs=[pl.BlockSpec((BLK, HID), lambda i: (i, 0)),
                   pl.BlockSpec((BLK, 1), lambda i: (i, 0)),
                   pl.BlockSpec((BLK, 1), lambda i: (i, 0))],
        out_shape=[jax.ShapeDtypeStruct((NP, HID), jnp.float32),
                   jax.ShapeDtypeStruct((NP, 1), jnp.float32),
                   jax.ShapeDtypeStruct((NP, 1), jnp.float32)],
    )(accf, accs, w2, aw2)


def _t3_body(pf_ref, ps_ref, w2_ref, o_ref):
    num = pf_ref[0] + pf_ref[1]
    ssum = (ps_ref[0] + ps_ref[1])[:, None]
    h = _hidden_from_acc(num, ssum)
    o = jnp.dot(h, w2_ref[...], preferred_element_type=jnp.float32)
    m = jnp.max(o, axis=1, keepdims=True)
    lse = jnp.log(jnp.sum(jnp.exp(o - m), axis=1, keepdims=True)) + m
    o_ref[...] = o - lse


def _t3(accf, accs, w2):
    return pl.pallas_call(
        _t3_body,
        grid=(NP // BLK,),
        in_specs=[pl.BlockSpec((2, BLK, HID), lambda i: (0, i, 0)),
                  pl.BlockSpec((2, BLK), lambda i: (0, i)),
                  pl.BlockSpec((HID, NCLS), lambda i: (0, 0))],
        out_specs=pl.BlockSpec((BLK, NCLS), lambda i: (i, 0)),
        out_shape=jax.ShapeDtypeStruct((NP, NCLS), jnp.float32),
    )(accf, accs, w2)


def kernel(local_features, edge_index, weight1, weight2,
           att_weight1, att_weight2):
    src = edge_index[0]
    dst = edge_index[1]
    xp = jnp.pad(local_features, ((0, NP - N), (0, 0)))
    srcp = jnp.pad(src, (0, EP - E), constant_values=TRASH).reshape(EPR, 128)
    dstp = jnp.pad(dst, (0, EP - E), constant_values=TRASH).reshape(EPR, 128)

    feat1, s1, d1 = _t1(xp, weight1, att_weight1)
    att1, mparts1 = _att_max(srcp, dstp, s1.reshape(NP), d1.reshape(NP))
    m1 = _combine(mparts1)
    accf1, accs1 = _agg(srcp, dstp, att1, m1, feat1)

    feat2, s2, d2 = _t2(accf1.reshape(2, NP, HID), accs1.reshape(2, NP),
                        weight2, att_weight2)
    att2, mparts2 = _att_max(srcp, dstp, s2.reshape(NP), d2.reshape(NP))
    m2 = _combine(mparts2)
    accf2, accs2 = _agg(srcp, dstp, att2, m2, feat2)

    out = _t3(accf2.reshape(2, NP, HID), accs2.reshape(2, NP), weight2)
    return out[:N]
